# baseline (speedup 1.0000x reference)
import jax
import jax.numpy as jnp
from jax import lax
from jax.experimental import pallas as pl
from jax.experimental.pallas import tpu as pltpu
from jax.experimental.pallas import tpu_sc as plsc

N = 10000
D = 128
E = 320000
NSUB = 16
CHUNK = 128
CPS = 160
WIN = 16
EPAD = NSUB * CPS * CHUNK
NPAD = 10240
ZROWS = NPAD // NSUB


def _fill_rows(rows_v, value):
    v16 = jnp.full((16,), value, jnp.float32)

    @pl.loop(0, rows_v.shape[0])
    def _(r):
        for c in range(D // 16):
            rows_v[r, pl.ds(c * 16, 16)] = v16


def _init_acc(acc_sh, rows_v, zb):
    for k in range(ZROWS // CHUNK):
        pltpu.sync_copy(rows_v, acc_sh.at[pl.ds(zb + k * CHUNK, CHUNK)])


def _copy_out(acc_sh, rows_v, out_ref, zb):
    for k in range(ZROWS // CHUNK):
        sl = pl.ds(zb + k * CHUNK, CHUNK)
        pltpu.sync_copy(acc_sh.at[sl], rows_v)
        pltpu.sync_copy(rows_v, out_ref.at[sl])


def _sc_segsum():
    mesh = plsc.VectorSubcoreMesh(core_axis_name="c", subcore_axis_name="s")
    f32 = jnp.float32
    out_type = [jax.ShapeDtypeStruct((NPAD, D), f32),
                jax.ShapeDtypeStruct((NPAD, D), f32)]
    scratch = [
        pltpu.VMEM((WIN, CHUNK), jnp.int32),
        pltpu.VMEM((WIN, CHUNK), jnp.int32),
        pltpu.VMEM((CHUNK, D), f32),
        pltpu.VMEM_SHARED((NPAD, D), f32),
    ]

    def body(table0, table1, src0, dst0, src1, dst1, agg0_o, agg1_o,
             idx_s, idx_d, rows_v, acc_sh):
        cid = lax.axis_index("c")
        sid = lax.axis_index("s")
        zb = sid * ZROWS

        _fill_rows(rows_v, 0.0)
        _init_acc(acc_sh, rows_v, zb)
        plsc.subcore_barrier()

        def phase(table, src2, dst2):
            for h in range(CPS // WIN):
                base = sid * CPS + h * WIN
                pltpu.sync_copy(src2.at[pl.ds(base, WIN)], idx_s)
                pltpu.sync_copy(dst2.at[pl.ds(base, WIN)], idx_d)

                @pl.loop(0, WIN)
                def _(j):
                    pltpu.sync_copy(table.at[idx_s.at[j]], rows_v)
                    pltpu.sync_copy(rows_v, acc_sh.at[idx_d.at[j]], add=True)

        @pl.when(cid == 0)
        def _():
            phase(table0, src0, dst0)

        @pl.when(cid == 1)
        def _():
            phase(table1, src1, dst1)

        plsc.subcore_barrier()

        @pl.when(cid == 0)
        def _():
            _copy_out(acc_sh, rows_v, agg0_o, zb)

        @pl.when(cid == 1)
        def _():
            _copy_out(acc_sh, rows_v, agg1_o, zb)

    return pl.kernel(body, out_type=out_type, mesh=mesh, scratch_types=scratch)


def _sc_counts():
    mesh = plsc.VectorSubcoreMesh(core_axis_name="c", subcore_axis_name="s")
    f32 = jnp.float32
    out_type = [jax.ShapeDtypeStruct((NPAD, D), f32),
                jax.ShapeDtypeStruct((NPAD, D), f32)]
    scratch = [
        pltpu.VMEM((WIN, CHUNK), jnp.int32),
        pltpu.VMEM((CHUNK, D), f32),
        pltpu.VMEM_SHARED((NPAD, D), f32),
    ]

    def body(dst0, dst1, cnt0_o, cnt1_o, idx_d, rows_v, acc_sh):
        cid = lax.axis_index("c")
        sid = lax.axis_index("s")
        zb = sid * ZROWS

        _fill_rows(rows_v, 0.0)
        _init_acc(acc_sh, rows_v, zb)
        plsc.subcore_barrier()
        _fill_rows(rows_v, 1.0)

        def phase(dst2):
            for h in range(CPS // WIN):
                base = sid * CPS + h * WIN
                pltpu.sync_copy(dst2.at[pl.ds(base, WIN)], idx_d)

                @pl.loop(0, WIN)
                def _(j):
                    pltpu.sync_copy(rows_v, acc_sh.at[idx_d.at[j]], add=True)

        @pl.when(cid == 0)
        def _():
            phase(dst0)

        @pl.when(cid == 1)
        def _():
            phase(dst1)

        plsc.subcore_barrier()

        @pl.when(cid == 0)
        def _():
            _copy_out(acc_sh, rows_v, cnt0_o, zb)

        @pl.when(cid == 1)
        def _():
            _copy_out(acc_sh, rows_v, cnt1_o, zb)

    return pl.kernel(body, out_type=out_type, mesh=mesh, scratch_types=scratch)


BLK = 512


def _tc_layer(xt, agg, cnt, wt, ws, wlin):
    def body(xt_r, agg_r, cnt_r, wt_r, ws_r, wlin_r, h_r, o_r):
        mean = agg_r[...] / jnp.maximum(cnt_r[:, 0:1], 1.0)
        h = (jnp.dot(xt_r[...], wt_r[...], precision=lax.Precision.HIGHEST,
                     preferred_element_type=jnp.float32)
             + jnp.dot(mean, ws_r[...], precision=lax.Precision.HIGHEST,
                       preferred_element_type=jnp.float32))
        h = jnp.maximum(h, 0.0)
        h_r[...] = h
        o_r[...] = jnp.dot(h, wlin_r[...], precision=lax.Precision.HIGHEST,
                           preferred_element_type=jnp.float32)

    return pl.pallas_call(
        body,
        grid=(NPAD // BLK,),
        in_specs=[
            pl.BlockSpec((BLK, D), lambda i: (i, 0)),
            pl.BlockSpec((BLK, D), lambda i: (i, 0)),
            pl.BlockSpec((BLK, D), lambda i: (i, 0)),
            pl.BlockSpec((D, D), lambda i: (0, 0)),
            pl.BlockSpec((D, D), lambda i: (0, 0)),
            pl.BlockSpec((D, D), lambda i: (0, 0)),
        ],
        out_specs=[
            pl.BlockSpec((BLK, D), lambda i: (i, 0)),
            pl.BlockSpec((BLK, D), lambda i: (i, 0)),
        ],
        out_shape=[
            jax.ShapeDtypeStruct((NPAD, D), jnp.float32),
            jax.ShapeDtypeStruct((NPAD, D), jnp.float32),
        ],
    )(xt, agg, cnt, wt, ws, wlin)


def _pad_rows(x):
    return jnp.concatenate(
        [x, jnp.zeros((NPAD - x.shape[0], x.shape[1]), x.dtype)])


def _prep_edges(edge):
    pad = EPAD - E
    src = jnp.concatenate([edge[0], jnp.zeros((pad,), jnp.int32)])
    dst = jnp.concatenate([edge[1], jnp.full((pad,), N, jnp.int32)])
    return (src.reshape(EPAD // CHUNK, CHUNK),
            dst.reshape(EPAD // CHUNK, CHUNK))


def _pad_lin(w):
    return jnp.zeros((D, D), jnp.float32).at[:, 0].set(w[:, 0])


def kernel(x_a, x_b, edge_ab, edge_ba, W_src1_ab, W_tgt1_ab, W_src1_ba, W_tgt1_ba,
           W_src2_ab, W_tgt2_ab, W_src2_ba, W_tgt2_ba, W_lin_a, W_lin_b,
           b_lin_a, b_lin_b):
    src_ab, dst_ab = _prep_edges(edge_ab)
    src_ba, dst_ba = _prep_edges(edge_ba)
    xa_p, xb_p = _pad_rows(x_a), _pad_rows(x_b)

    cnt_ab, cnt_ba = _sc_counts()(dst_ab, dst_ba)

    agg_ab, agg_ba = _sc_segsum()(x_a, x_b, src_ab, dst_ab, src_ba, dst_ba)
    hb1, _ = _tc_layer(xb_p, agg_ab, cnt_ab, W_tgt1_ab, W_src1_ab,
                       jnp.zeros((D, D), jnp.float32))
    ha1, _ = _tc_layer(xa_p, agg_ba, cnt_ba, W_tgt1_ba, W_src1_ba,
                       jnp.zeros((D, D), jnp.float32))

    agg_ab2, agg_ba2 = _sc_segsum()(ha1, hb1, src_ab, dst_ab, src_ba, dst_ba)
    hb2, ob = _tc_layer(hb1, agg_ab2, cnt_ab, W_tgt2_ab, W_src2_ab,
                        _pad_lin(W_lin_b))
    ha2, oa = _tc_layer(ha1, agg_ba2, cnt_ba, W_tgt2_ba, W_src2_ba,
                        _pad_lin(W_lin_a))

    out_a = oa[:N, 0:1] + b_lin_a
    out_b = ob[:N, 0:1] + b_lin_b
    return (ha2[:N], hb2[:N], out_a, out_b)


# speedup vs baseline: 1.0064x; 1.0064x over previous
import jax
import jax.numpy as jnp
from jax import lax
from jax.experimental import pallas as pl
from jax.experimental.pallas import tpu as pltpu
from jax.experimental.pallas import tpu_sc as plsc

N = 10000
D = 128
E = 320000
NSUB = 16
CHUNK = 128
CPS = 160
WIN = 32
EPAD = NSUB * CPS * CHUNK
NPAD = 10240
ZROWS = NPAD // NSUB


def _fill_rows(rows_v, value):
    v16 = jnp.full((16,), value, jnp.float32)

    @pl.loop(0, rows_v.shape[0])
    def _(r):
        for c in range(D // 16):
            rows_v[r, pl.ds(c * 16, 16)] = v16


def _init_acc(acc_sh, rows_v, zb):
    for k in range(ZROWS // CHUNK):
        pltpu.sync_copy(rows_v, acc_sh.at[pl.ds(zb + k * CHUNK, CHUNK)])


def _copy_out(acc_sh, rows_v, out_ref, zb):
    for k in range(ZROWS // CHUNK):
        sl = pl.ds(zb + k * CHUNK, CHUNK)
        pltpu.sync_copy(acc_sh.at[sl], rows_v)
        pltpu.sync_copy(rows_v, out_ref.at[sl])


def _sc_segsum():
    mesh = plsc.VectorSubcoreMesh(core_axis_name="c", subcore_axis_name="s")
    f32 = jnp.float32
    out_type = [jax.ShapeDtypeStruct((NPAD, D), f32),
                jax.ShapeDtypeStruct((NPAD, D), f32)]
    scratch = [
        pltpu.VMEM((WIN, CHUNK), jnp.int32),
        pltpu.VMEM((WIN, CHUNK), jnp.int32),
        pltpu.VMEM((CHUNK, D), f32),
        pltpu.VMEM_SHARED((NPAD, D), f32),
    ]

    def body(table0, table1, src0, dst0, src1, dst1, agg0_o, agg1_o,
             idx_s, idx_d, rows_v, acc_sh):
        cid = lax.axis_index("c")
        sid = lax.axis_index("s")
        zb = sid * ZROWS

        _fill_rows(rows_v, 0.0)
        _init_acc(acc_sh, rows_v, zb)
        plsc.subcore_barrier()

        def phase(table, src2, dst2):
            for h in range(CPS // WIN):
                base = sid * CPS + h * WIN
                pltpu.sync_copy(src2.at[pl.ds(base, WIN)], idx_s)
                pltpu.sync_copy(dst2.at[pl.ds(base, WIN)], idx_d)

                @pl.loop(0, WIN)
                def _(j):
                    pltpu.sync_copy(table.at[idx_s.at[j]], rows_v)
                    pltpu.sync_copy(rows_v, acc_sh.at[idx_d.at[j]], add=True)

        @pl.when(cid == 0)
        def _():
            phase(table0, src0, dst0)

        @pl.when(cid == 1)
        def _():
            phase(table1, src1, dst1)

        plsc.subcore_barrier()

        @pl.when(cid == 0)
        def _():
            _copy_out(acc_sh, rows_v, agg0_o, zb)

        @pl.when(cid == 1)
        def _():
            _copy_out(acc_sh, rows_v, agg1_o, zb)

    return pl.kernel(body, out_type=out_type, mesh=mesh, scratch_types=scratch)


def _sc_counts():
    mesh = plsc.VectorSubcoreMesh(core_axis_name="c", subcore_axis_name="s")
    f32 = jnp.float32
    out_type = [jax.ShapeDtypeStruct((NPAD, D), f32),
                jax.ShapeDtypeStruct((NPAD, D), f32)]
    scratch = [
        pltpu.VMEM((WIN, CHUNK), jnp.int32),
        pltpu.VMEM((CHUNK, D), f32),
        pltpu.VMEM_SHARED((NPAD, D), f32),
    ]

    def body(dst0, dst1, cnt0_o, cnt1_o, idx_d, rows_v, acc_sh):
        cid = lax.axis_index("c")
        sid = lax.axis_index("s")
        zb = sid * ZROWS

        _fill_rows(rows_v, 0.0)
        _init_acc(acc_sh, rows_v, zb)
        plsc.subcore_barrier()
        _fill_rows(rows_v, 1.0)

        def phase(dst2):
            for h in range(CPS // WIN):
                base = sid * CPS + h * WIN
                pltpu.sync_copy(dst2.at[pl.ds(base, WIN)], idx_d)

                @pl.loop(0, WIN)
                def _(j):
                    pltpu.sync_copy(rows_v, acc_sh.at[idx_d.at[j]], add=True)

        @pl.when(cid == 0)
        def _():
            phase(dst0)

        @pl.when(cid == 1)
        def _():
            phase(dst1)

        plsc.subcore_barrier()

        @pl.when(cid == 0)
        def _():
            _copy_out(acc_sh, rows_v, cnt0_o, zb)

        @pl.when(cid == 1)
        def _():
            _copy_out(acc_sh, rows_v, cnt1_o, zb)

    return pl.kernel(body, out_type=out_type, mesh=mesh, scratch_types=scratch)


BLK = 512


def _tc_layer(xt, agg, cnt, wt, ws, wlin=None):
    with_lin = wlin is not None

    def body(*refs):
        if with_lin:
            xt_r, agg_r, cnt_r, wt_r, ws_r, wlin_r, h_r, o_r = refs
        else:
            xt_r, agg_r, cnt_r, wt_r, ws_r, h_r = refs
        mean = agg_r[...] / jnp.maximum(cnt_r[:, 0:1], 1.0)
        h = (jnp.dot(xt_r[...], wt_r[...], precision=lax.Precision.HIGHEST,
                     preferred_element_type=jnp.float32)
             + jnp.dot(mean, ws_r[...], precision=lax.Precision.HIGHEST,
                       preferred_element_type=jnp.float32))
        h = jnp.maximum(h, 0.0)
        h_r[...] = h
        if with_lin:
            o_r[...] = jnp.dot(h, wlin_r[...], precision=lax.Precision.HIGHEST,
                               preferred_element_type=jnp.float32)

    row_spec = pl.BlockSpec((BLK, D), lambda i: (i, 0))
    w_spec = pl.BlockSpec((D, D), lambda i: (0, 0))
    in_specs = [row_spec, row_spec, row_spec, w_spec, w_spec]
    args = [xt, agg, cnt, wt, ws]
    out_specs, out_shape = row_spec, jax.ShapeDtypeStruct((NPAD, D), jnp.float32)
    if with_lin:
        in_specs.append(w_spec)
        args.append(wlin)
        out_specs, out_shape = [out_specs] * 2, [out_shape] * 2
    return pl.pallas_call(
        body,
        grid=(NPAD // BLK,),
        in_specs=in_specs,
        out_specs=out_specs,
        out_shape=out_shape,
    )(*args)


def _pad_rows(x):
    return jnp.concatenate(
        [x, jnp.zeros((NPAD - x.shape[0], x.shape[1]), x.dtype)])


def _prep_edges(edge):
    pad = EPAD - E
    src = jnp.concatenate([edge[0], jnp.zeros((pad,), jnp.int32)])
    dst = jnp.concatenate([edge[1], jnp.full((pad,), N, jnp.int32)])
    return (src.reshape(EPAD // CHUNK, CHUNK),
            dst.reshape(EPAD // CHUNK, CHUNK))


def _pad_lin(w):
    return jnp.zeros((D, D), jnp.float32).at[:, 0].set(w[:, 0])


def kernel(x_a, x_b, edge_ab, edge_ba, W_src1_ab, W_tgt1_ab, W_src1_ba, W_tgt1_ba,
           W_src2_ab, W_tgt2_ab, W_src2_ba, W_tgt2_ba, W_lin_a, W_lin_b,
           b_lin_a, b_lin_b):
    src_ab, dst_ab = _prep_edges(edge_ab)
    src_ba, dst_ba = _prep_edges(edge_ba)
    xa_p, xb_p = _pad_rows(x_a), _pad_rows(x_b)

    cnt_ab, cnt_ba = _sc_counts()(dst_ab, dst_ba)

    agg_ab, agg_ba = _sc_segsum()(x_a, x_b, src_ab, dst_ab, src_ba, dst_ba)
    hb1 = _tc_layer(xb_p, agg_ab, cnt_ab, W_tgt1_ab, W_src1_ab)
    ha1 = _tc_layer(xa_p, agg_ba, cnt_ba, W_tgt1_ba, W_src1_ba)

    agg_ab2, agg_ba2 = _sc_segsum()(ha1, hb1, src_ab, dst_ab, src_ba, dst_ba)
    hb2, ob = _tc_layer(hb1, agg_ab2, cnt_ab, W_tgt2_ab, W_src2_ab,
                        _pad_lin(W_lin_b))
    ha2, oa = _tc_layer(ha1, agg_ba2, cnt_ba, W_tgt2_ba, W_src2_ba,
                        _pad_lin(W_lin_a))

    out_a = oa[:N, 0:1] + b_lin_a
    out_b = ob[:N, 0:1] + b_lin_b
    return (ha2[:N], hb2[:N], out_a, out_b)
